# baseline (speedup 1.0000x reference)
import jax
import jax.numpy as jnp
from jax import lax
from jax.experimental import pallas as pl
from jax.experimental.pallas import tpu as pltpu

BATCH = 64
VALUE_DIM = 64
BN = 20000


def _retrieve_body(q_ref, v_ref, o_ref, acc_ref, m_ref, l_ref):
    i = pl.program_id(0)
    nb = pl.num_programs(0)

    @pl.when(i == 0)
    def _init():
        acc_ref[...] = jnp.zeros_like(acc_ref)
        m_ref[...] = jnp.full_like(m_ref, -jnp.inf)
        l_ref[...] = jnp.zeros_like(l_ref)

    q = q_ref[...]
    v = v_ref[...]
    logits = lax.dot_general(q, v, (((1,), (1,)), ((), ())),
                             preferred_element_type=jnp.float32)
    m_prev = m_ref[...]
    m_new = jnp.maximum(m_prev, jnp.max(logits, axis=1, keepdims=True))
    corr = jnp.exp(m_prev - m_new)
    p = jnp.exp(logits - m_new)
    m_ref[...] = m_new
    l_ref[...] = l_ref[...] * corr + jnp.sum(p, axis=1, keepdims=True)
    acc_ref[...] = acc_ref[...] * corr + lax.dot_general(
        p.astype(jnp.bfloat16), v.astype(jnp.bfloat16),
        (((1,), (0,)), ((), ())), preferred_element_type=jnp.float32)

    @pl.when(i == nb - 1)
    def _fin():
        o_ref[...] = acc_ref[...] / l_ref[...]


@jax.jit
def kernel(query, values):
    cap = values.shape[0]
    nb = cap // BN
    assert nb * BN == cap
    one = jnp.float32(1.0) + jnp.float32(0.0) * query[0, 0]
    return pl.pallas_call(
        _retrieve_body,
        grid=(nb,),
        in_specs=[
            pl.BlockSpec((BATCH, VALUE_DIM), lambda i: (0, 0)),
            pl.BlockSpec((BN, VALUE_DIM), lambda i: (i, 0)),
        ],
        out_specs=pl.BlockSpec((BATCH, VALUE_DIM), lambda i: (0, 0)),
        out_shape=jax.ShapeDtypeStruct((BATCH, VALUE_DIM), jnp.float32),
        scratch_shapes=[
            pltpu.VMEM((BATCH, VALUE_DIM), jnp.float32),
            pltpu.VMEM((BATCH, 1), jnp.float32),
            pltpu.VMEM((BATCH, 1), jnp.float32),
        ],
        compiler_params=pltpu.CompilerParams(
            allow_input_fusion=[False, True]),
    )(query, values * one)


# speedup vs baseline: 1.4788x; 1.4788x over previous
import jax
import jax.numpy as jnp
from jax import lax
from jax.experimental import pallas as pl
from jax.experimental.pallas import tpu as pltpu

BATCH = 64
VALUE_DIM = 64
BN = 5000


def _update(q, v, acc_ref, m_ref, l_ref):
    logits = lax.dot_general(q, v, (((1,), (1,)), ((), ())),
                             preferred_element_type=jnp.float32)
    m_prev = m_ref[...]
    m_new = jnp.maximum(m_prev, jnp.max(logits, axis=1, keepdims=True))
    corr = jnp.exp(m_prev - m_new)
    p = jnp.exp(logits - m_new)
    m_ref[...] = m_new
    l_ref[...] = l_ref[...] * corr + jnp.sum(p, axis=1, keepdims=True)
    acc_ref[...] = acc_ref[...] * corr + lax.dot_general(
        p.astype(jnp.bfloat16), v.astype(jnp.bfloat16),
        (((1,), (0,)), ((), ())), preferred_element_type=jnp.float32)


def _retrieve_body(q_ref, v1_ref, v2_ref, o_ref, acc_ref, m_ref, l_ref):
    i = pl.program_id(0)
    nb = pl.num_programs(0)

    @pl.when(i == 0)
    def _init():
        acc_ref[...] = jnp.zeros_like(acc_ref)
        m_ref[...] = jnp.full_like(m_ref, -jnp.inf)
        l_ref[...] = jnp.zeros_like(l_ref)

    q = q_ref[...]
    _update(q, v1_ref[...], acc_ref, m_ref, l_ref)
    _update(q, v2_ref[...], acc_ref, m_ref, l_ref)

    @pl.when(i == nb - 1)
    def _fin():
        o_ref[...] = acc_ref[...] / l_ref[...]


@jax.jit
def kernel(query, values):
    cap = values.shape[0]
    nb = cap // (2 * BN)
    assert nb * 2 * BN == cap
    one = jnp.float32(1.0) + jnp.float32(0.0) * query[0, 0]
    v = values * one
    return pl.pallas_call(
        _retrieve_body,
        grid=(nb,),
        in_specs=[
            pl.BlockSpec((BATCH, VALUE_DIM), lambda i: (0, 0)),
            pl.BlockSpec((BN, VALUE_DIM), lambda i: (i, 0)),
            pl.BlockSpec((BN, VALUE_DIM), lambda i, _nb=nb: (_nb + i, 0)),
        ],
        out_specs=pl.BlockSpec((BATCH, VALUE_DIM), lambda i: (0, 0)),
        out_shape=jax.ShapeDtypeStruct((BATCH, VALUE_DIM), jnp.float32),
        scratch_shapes=[
            pltpu.VMEM((BATCH, VALUE_DIM), jnp.float32),
            pltpu.VMEM((BATCH, 1), jnp.float32),
            pltpu.VMEM((BATCH, 1), jnp.float32),
        ],
        compiler_params=pltpu.CompilerParams(
            allow_input_fusion=[False, True, True]),
    )(query, v, v)


# speedup vs baseline: 1.5265x; 1.0322x over previous
import jax
import jax.numpy as jnp
from jax import lax
from jax.experimental import pallas as pl
from jax.experimental.pallas import tpu as pltpu

BATCH = 64
VALUE_DIM = 64
BN = 10000


def _retrieve_body(q_ref, v_ref, o_ref, acc_ref, m_ref, l_ref):
    i = pl.program_id(0)
    nb = pl.num_programs(0)

    @pl.when(i == 0)
    def _init():
        acc_ref[...] = jnp.zeros_like(acc_ref)
        m_ref[...] = jnp.full_like(m_ref, -jnp.inf)
        l_ref[...] = jnp.zeros_like(l_ref)

    q = q_ref[...]
    v = v_ref[...]
    logits = lax.dot_general(q, v, (((1,), (1,)), ((), ())),
                             preferred_element_type=jnp.float32)
    m_prev = m_ref[...]
    m_new = jnp.maximum(m_prev, jnp.max(logits, axis=1, keepdims=True))
    corr = jnp.exp(m_prev - m_new)
    p = jnp.exp(logits - m_new)
    m_ref[...] = m_new
    l_ref[...] = l_ref[...] * corr + jnp.sum(p, axis=1, keepdims=True)
    acc_ref[...] = acc_ref[...] * corr + lax.dot_general(
        p.astype(jnp.bfloat16), v.astype(jnp.bfloat16),
        (((1,), (0,)), ((), ())), preferred_element_type=jnp.float32)

    @pl.when(i == nb - 1)
    def _fin():
        o_ref[...] = acc_ref[...] / l_ref[...]


@jax.jit
def kernel(query, values):
    cap = values.shape[0]
    nb = cap // BN
    assert nb * BN == cap
    one = jnp.float32(1.0) + jnp.float32(0.0) * query[0, 0]
    return pl.pallas_call(
        _retrieve_body,
        grid=(nb,),
        in_specs=[
            pl.BlockSpec((BATCH, VALUE_DIM), lambda i: (0, 0)),
            pl.BlockSpec((BN, VALUE_DIM), lambda i: (i, 0)),
        ],
        out_specs=pl.BlockSpec((BATCH, VALUE_DIM), lambda i: (0, 0)),
        out_shape=jax.ShapeDtypeStruct((BATCH, VALUE_DIM), jnp.float32),
        scratch_shapes=[
            pltpu.VMEM((BATCH, VALUE_DIM), jnp.float32),
            pltpu.VMEM((BATCH, 1), jnp.float32),
            pltpu.VMEM((BATCH, 1), jnp.float32),
        ],
        compiler_params=pltpu.CompilerParams(
            allow_input_fusion=[False, True]),
    )(query, values * one)
